# Initial kernel scaffold; baseline (speedup 1.0000x reference)
import jax
import jax.numpy as jnp
from jax import lax
from jax.experimental import pallas as pl
from jax.experimental.pallas import tpu as pltpu
from jax.experimental.pallas import tpu_sc as plsc

N_NODES = 10000
N_PAD = 10240
N_EDGES = 320000
N_WORKERS = 32
E_PER_W = N_EDGES // N_WORKERS
CHUNK = 80
N_CHUNKS = E_PER_W // CHUNK
ROWS_T = N_PAD // 16

_MESH = plsc.VectorSubcoreMesh(core_axis_name="c", subcore_axis_name="s")


def _deg_body(dst_hbm, ones_hbm, zero_hbm, out_hbm, dstv, onesv, stage, acc):
    c = lax.axis_index("c")
    s = lax.axis_index("s")
    wid = c * 16 + s
    pltpu.sync_copy(zero_hbm, stage)
    pltpu.sync_copy(stage, acc.at[pl.ds(s * ROWS_T, ROWS_T)])
    pltpu.sync_copy(dst_hbm.at[wid], dstv)
    pltpu.sync_copy(ones_hbm, onesv)
    plsc.subcore_barrier()

    def chunk(j, carry):
        pltpu.sync_copy(onesv, acc.at[dstv.at[j]], add=True)
        return carry

    lax.fori_loop(0, N_CHUNKS, chunk, 0)
    plsc.subcore_barrier()
    pltpu.sync_copy(acc.at[pl.ds(s * ROWS_T, ROWS_T)], stage)
    pltpu.sync_copy(stage, out_hbm.at[c, pl.ds(s * ROWS_T, ROWS_T)])


_deg_kernel = pl.kernel(
    _deg_body,
    out_type=jax.ShapeDtypeStruct((2, N_PAD, 1), jnp.float32),
    mesh=_MESH,
    scratch_types=[
        pltpu.VMEM((N_CHUNKS, CHUNK), jnp.int32),
        pltpu.VMEM((CHUNK, 1), jnp.float32),
        pltpu.VMEM((ROWS_T, 1), jnp.float32),
        pltpu.VMEM_SHARED((N_PAD, 1), jnp.float32),
    ],
)


def _agg_body(src_hbm, dst_hbm, p_hbm, out_hbm, srcv, dstv, rows, stage, acc,
              sem):
    c = lax.axis_index("c")
    s = lax.axis_index("s")
    wid = c * 16 + s
    pltpu.sync_copy(p_hbm.at[pl.ds(s * ROWS_T, ROWS_T)], stage)
    pltpu.sync_copy(stage, acc.at[pl.ds(s * ROWS_T, ROWS_T)])
    pltpu.sync_copy(src_hbm.at[wid], srcv)
    pltpu.sync_copy(dst_hbm.at[wid], dstv)
    plsc.subcore_barrier()

    def chunk(j, carry):
        pltpu.async_copy(p_hbm.at[srcv.at[j]], rows, sem).wait()
        pltpu.sync_copy(rows, acc.at[dstv.at[j]], add=True)
        return carry

    lax.fori_loop(0, N_CHUNKS, chunk, 0)
    plsc.subcore_barrier()
    pltpu.sync_copy(acc.at[pl.ds(s * ROWS_T, ROWS_T)], stage)
    pltpu.sync_copy(stage, out_hbm.at[c, pl.ds(s * ROWS_T, ROWS_T)])


def _make_agg(width):
    return pl.kernel(
        _agg_body,
        out_type=jax.ShapeDtypeStruct((2, N_PAD, width), jnp.float32),
        mesh=_MESH,
        scratch_types=[
            pltpu.VMEM((N_CHUNKS, CHUNK), jnp.int32),
            pltpu.VMEM((N_CHUNKS, CHUNK), jnp.int32),
            pltpu.VMEM((CHUNK, width), jnp.float32),
            pltpu.VMEM((ROWS_T, width), jnp.float32),
            pltpu.VMEM_SHARED((N_PAD, width), jnp.float32),
            pltpu.SemaphoreType.DMA,
        ],
    )


_agg20 = _make_agg(20)
_agg10 = _make_agg(10)


def _tc1_body(deg2_ref, x_ref, w1_ref, p1_ref, dinv_ref):
    deg = deg2_ref[0] + deg2_ref[1] + 1.0
    dinv = lax.rsqrt(deg)
    h = jnp.dot(x_ref[...], w1_ref[...], preferred_element_type=jnp.float32)
    p1_ref[...] = h * dinv
    dinv_ref[...] = dinv


def _tc2_body(acc_ref, p1_ref, dinv_ref, b1_ref, wmu_ref, p2_ref):
    q1 = acc_ref[0] + acc_ref[1] - p1_ref[...]
    h1 = jnp.maximum(q1 * dinv_ref[...] + b1_ref[...], 0.0)
    h2 = jnp.dot(h1, wmu_ref[...], preferred_element_type=jnp.float32)
    p2_ref[...] = h2 * dinv_ref[...]


def _tc3_body(acc_ref, p2_ref, dinv_ref, bmu_ref, z_ref):
    z_ref[...] = (acc_ref[0] + acc_ref[1] - p2_ref[...]) * dinv_ref[...] \
        + bmu_ref[...]


@jax.jit
def kernel(x, edge_index, W1, b1, W_mu, b_mu, W_ls, b_ls):
    del W_ls, b_ls
    ei = edge_index.astype(jnp.int32)
    src_r = ei[0].reshape(N_WORKERS, N_CHUNKS, CHUNK)
    dst_r = ei[1].reshape(N_WORKERS, N_CHUNKS, CHUNK)
    x_pad = jnp.zeros((N_PAD, x.shape[1]), x.dtype).at[:N_NODES].set(x)
    ones_c = jnp.ones((CHUNK, 1), jnp.float32)
    zero_t = jnp.zeros((ROWS_T, 1), jnp.float32)

    deg2 = _deg_kernel(dst_r, ones_c, zero_t)

    p1, dinv = pl.pallas_call(
        _tc1_body,
        out_shape=(jax.ShapeDtypeStruct((N_PAD, 20), jnp.float32),
                   jax.ShapeDtypeStruct((N_PAD, 1), jnp.float32)),
    )(deg2, x_pad, W1)

    acc1 = _agg20(src_r, dst_r, p1)

    p2 = pl.pallas_call(
        _tc2_body,
        out_shape=jax.ShapeDtypeStruct((N_PAD, 10), jnp.float32),
    )(acc1, p1, dinv, b1.reshape(1, 20), W_mu)

    acc2 = _agg10(src_r, dst_r, p2)

    z = pl.pallas_call(
        _tc3_body,
        out_shape=jax.ShapeDtypeStruct((N_PAD, 10), jnp.float32),
    )(acc2, p2, dinv, b_mu.reshape(1, 10))

    return z[:N_NODES]


# speedup vs baseline: 28.6356x; 28.6356x over previous
import jax
import jax.numpy as jnp
from jax import lax
from jax.experimental import pallas as pl
from jax.experimental.pallas import tpu as pltpu
from jax.experimental.pallas import tpu_sc as plsc

N_NODES = 10000
N_PAD = 10240
N_EDGES = 320000
N_WORKERS = 32
E_PER_W = N_EDGES // N_WORKERS
CHUNK = 80
N_CHUNKS = E_PER_W // CHUNK
ROWS_T = N_PAD // 16

_MESH = plsc.VectorSubcoreMesh(core_axis_name="c", subcore_axis_name="s")
_SC_PARAMS = pltpu.CompilerParams(use_tc_tiling_on_sc=False)


def _deg_body(dst_hbm, ones_hbm, zero_hbm, out_hbm, dstv, onesv, stage, acc):
    c = lax.axis_index("c")
    s = lax.axis_index("s")
    wid = c * 16 + s
    pltpu.sync_copy(zero_hbm, stage)
    pltpu.sync_copy(stage, acc.at[pl.ds(s * ROWS_T, ROWS_T)])
    pltpu.sync_copy(dst_hbm.at[wid], dstv)
    pltpu.sync_copy(ones_hbm, onesv)
    plsc.subcore_barrier()

    def chunk(j, carry):
        pltpu.sync_copy(onesv, acc.at[dstv.at[j]], add=True)
        return carry

    lax.fori_loop(0, N_CHUNKS, chunk, 0)
    plsc.subcore_barrier()
    pltpu.sync_copy(acc.at[pl.ds(s * ROWS_T, ROWS_T)], stage)
    pltpu.sync_copy(stage, out_hbm.at[c, pl.ds(s * ROWS_T, ROWS_T)])


_deg_kernel = pl.kernel(
    _deg_body,
    out_type=jax.ShapeDtypeStruct((2, N_PAD, 1), jnp.float32),
    mesh=_MESH,
    scratch_types=[
        pltpu.VMEM((N_CHUNKS, CHUNK), jnp.int32),
        pltpu.VMEM((CHUNK, 1), jnp.float32),
        pltpu.VMEM((ROWS_T, 1), jnp.float32),
        pltpu.VMEM_SHARED((N_PAD, 1), jnp.float32),
    ],
    compiler_params=_SC_PARAMS,
)


def _agg_body(src_hbm, dst_hbm, p_hbm, out_hbm, srcv, dstv, rows, stage, acc,
              sem):
    c = lax.axis_index("c")
    s = lax.axis_index("s")
    wid = c * 16 + s
    pltpu.sync_copy(p_hbm.at[pl.ds(s * ROWS_T, ROWS_T)], stage)
    pltpu.sync_copy(stage, acc.at[pl.ds(s * ROWS_T, ROWS_T)])
    pltpu.sync_copy(src_hbm.at[wid], srcv)
    pltpu.sync_copy(dst_hbm.at[wid], dstv)
    plsc.subcore_barrier()

    def chunk(j, carry):
        pltpu.async_copy(p_hbm.at[srcv.at[j]], rows, sem).wait()
        pltpu.sync_copy(rows, acc.at[dstv.at[j]], add=True)
        return carry

    lax.fori_loop(0, N_CHUNKS, chunk, 0)
    plsc.subcore_barrier()
    pltpu.sync_copy(acc.at[pl.ds(s * ROWS_T, ROWS_T)], stage)
    pltpu.sync_copy(stage, out_hbm.at[c, pl.ds(s * ROWS_T, ROWS_T)])


def _make_agg(width):
    return pl.kernel(
        _agg_body,
        out_type=jax.ShapeDtypeStruct((2, N_PAD, width), jnp.float32),
        mesh=_MESH,
        scratch_types=[
            pltpu.VMEM((N_CHUNKS, CHUNK), jnp.int32),
            pltpu.VMEM((N_CHUNKS, CHUNK), jnp.int32),
            pltpu.VMEM((CHUNK, width), jnp.float32),
            pltpu.VMEM((ROWS_T, width), jnp.float32),
            pltpu.VMEM_SHARED((N_PAD, width), jnp.float32),
            pltpu.SemaphoreType.DMA,
        ],
        compiler_params=_SC_PARAMS,
    )


_agg20 = _make_agg(20)
_agg10 = _make_agg(10)


def _tc1_body(deg2_ref, x_ref, w1_ref, p1_ref, dinv_ref):
    deg = deg2_ref[0] + deg2_ref[1] + 1.0
    dinv = lax.rsqrt(deg)
    h = jnp.dot(x_ref[...], w1_ref[...], preferred_element_type=jnp.float32)
    p1_ref[...] = h * dinv
    dinv_ref[...] = dinv


def _tc2_body(acc_ref, p1_ref, dinv_ref, b1_ref, wmu_ref, p2_ref):
    q1 = acc_ref[0] + acc_ref[1] - p1_ref[...]
    h1 = jnp.maximum(q1 * dinv_ref[...] + b1_ref[...], 0.0)
    h2 = jnp.dot(h1, wmu_ref[...], preferred_element_type=jnp.float32)
    p2_ref[...] = h2 * dinv_ref[...]


def _tc3_body(acc_ref, p2_ref, dinv_ref, bmu_ref, z_ref):
    z_ref[...] = (acc_ref[0] + acc_ref[1] - p2_ref[...]) * dinv_ref[...] \
        + bmu_ref[...]


@jax.jit
def kernel(x, edge_index, W1, b1, W_mu, b_mu, W_ls, b_ls):
    del W_ls, b_ls
    ei = edge_index.astype(jnp.int32)
    src_r = ei[0].reshape(N_WORKERS, N_CHUNKS, CHUNK)
    dst_r = ei[1].reshape(N_WORKERS, N_CHUNKS, CHUNK)
    x_pad = jnp.zeros((N_PAD, x.shape[1]), x.dtype).at[:N_NODES].set(x)
    ones_c = jnp.ones((CHUNK, 1), jnp.float32)
    zero_t = jnp.zeros((ROWS_T, 1), jnp.float32)

    deg2 = _deg_kernel(dst_r, ones_c, zero_t)

    p1, dinv = pl.pallas_call(
        _tc1_body,
        out_shape=(jax.ShapeDtypeStruct((N_PAD, 20), jnp.float32),
                   jax.ShapeDtypeStruct((N_PAD, 1), jnp.float32)),
    )(deg2, x_pad, W1)

    acc1 = _agg20(src_r, dst_r, p1)

    p2 = pl.pallas_call(
        _tc2_body,
        out_shape=jax.ShapeDtypeStruct((N_PAD, 10), jnp.float32),
    )(acc1, p1, dinv, b1.reshape(1, 20), W_mu)

    acc2 = _agg10(src_r, dst_r, p2)

    z = pl.pallas_call(
        _tc3_body,
        out_shape=jax.ShapeDtypeStruct((N_PAD, 10), jnp.float32),
    )(acc2, p2, dinv, b_mu.reshape(1, 10))

    return z[:N_NODES]


# speedup vs baseline: 36.4842x; 1.2741x over previous
import jax
import jax.numpy as jnp
from jax import lax
from jax.experimental import pallas as pl
from jax.experimental.pallas import tpu as pltpu
from jax.experimental.pallas import tpu_sc as plsc

N_NODES = 10000
N_PAD = 10240
PAD_NODE = N_PAD - 1
N_EDGES = 320000
N_WORKERS = 32
E_PER_W = 10240
CHUNK = 128
N_CHUNKS = E_PER_W // CHUNK
RING = 5
N_ROUNDS = N_CHUNKS // RING
ROWS_T = N_PAD // 16

_MESH = plsc.VectorSubcoreMesh(core_axis_name="c", subcore_axis_name="s")
_SC_PARAMS = pltpu.CompilerParams(use_tc_tiling_on_sc=False)


def _deg_body(dst_hbm, ones_hbm, zero_hbm, out_hbm, dstv, onesv, stage, acc,
              ssem):
    c = lax.axis_index("c")
    s = lax.axis_index("s")
    wid = c * 16 + s
    pltpu.sync_copy(zero_hbm, stage)
    pltpu.sync_copy(stage, acc.at[pl.ds(s * ROWS_T, ROWS_T)])
    pltpu.sync_copy(dst_hbm.at[wid], dstv)
    pltpu.sync_copy(ones_hbm, onesv)
    plsc.subcore_barrier()

    def s_start(q, b):
        pltpu.async_copy(onesv, acc.at[dstv.at[q]], ssem.at[b], add=True)

    def s_wait(q, b):
        pltpu.make_async_copy(onesv, acc.at[dstv.at[q]], ssem.at[b]).wait()

    def round_body(r, carry):
        s_start(r, 0)
        s_wait(r, 0)
        return carry

    lax.fori_loop(0, N_CHUNKS, round_body, 0)
    plsc.subcore_barrier()
    pltpu.sync_copy(acc.at[pl.ds(s * ROWS_T, ROWS_T)], stage)
    pltpu.sync_copy(stage, out_hbm.at[c, pl.ds(s * ROWS_T, ROWS_T)])


_deg_kernel = pl.kernel(
    _deg_body,
    out_type=jax.ShapeDtypeStruct((2, N_PAD, 1), jnp.float32),
    mesh=_MESH,
    scratch_types=[
        pltpu.VMEM((N_CHUNKS, CHUNK), jnp.int32),
        pltpu.VMEM((CHUNK, 1), jnp.float32),
        pltpu.VMEM((ROWS_T, 1), jnp.float32),
        pltpu.VMEM_SHARED((N_PAD, 1), jnp.float32),
        pltpu.SemaphoreType.DMA((RING,)),
    ],
    compiler_params=_SC_PARAMS,
)


def _agg_body(src_hbm, dst_hbm, p_hbm, out_hbm, srcv, dstv, rows, stage, acc,
              gsem, ssem):
    c = lax.axis_index("c")
    s = lax.axis_index("s")
    wid = c * 16 + s
    pltpu.sync_copy(p_hbm.at[pl.ds(s * ROWS_T, ROWS_T)], stage)
    pltpu.sync_copy(stage, acc.at[pl.ds(s * ROWS_T, ROWS_T)])
    pltpu.sync_copy(src_hbm.at[wid], srcv)
    pltpu.sync_copy(dst_hbm.at[wid], dstv)
    plsc.subcore_barrier()

    def g_start(q, b):
        pltpu.async_copy(p_hbm.at[srcv.at[q]], rows.at[b], gsem.at[b])

    def g_wait(q, b):
        pltpu.make_async_copy(p_hbm.at[srcv.at[q]], rows.at[b],
                              gsem.at[b]).wait()

    def s_start(q, b):
        pltpu.async_copy(rows.at[b], acc.at[dstv.at[q]], ssem.at[b], add=True)

    def s_wait(q, b):
        pltpu.make_async_copy(rows.at[b], acc.at[dstv.at[q]],
                              ssem.at[b]).wait()

    for b in range(RING):
        g_start(b, b)

    def round_body(r, carry):
        q0 = r * RING
        for b in range(RING):
            q = q0 + b
            g_wait(q, b)
            s_start(q, b)
            s_wait(q, b)
            g_start(q + RING, b)
        return carry

    lax.fori_loop(0, N_ROUNDS - 1, round_body, 0)
    qL = (N_ROUNDS - 1) * RING
    for b in range(RING):
        g_wait(qL + b, b)
        s_start(qL + b, b)
        s_wait(qL + b, b)
    plsc.subcore_barrier()
    pltpu.sync_copy(acc.at[pl.ds(s * ROWS_T, ROWS_T)], stage)
    pltpu.sync_copy(stage, out_hbm.at[c, pl.ds(s * ROWS_T, ROWS_T)])


def _make_agg(width):
    return pl.kernel(
        _agg_body,
        out_type=jax.ShapeDtypeStruct((2, N_PAD, width), jnp.float32),
        mesh=_MESH,
        scratch_types=[
            pltpu.VMEM((N_CHUNKS, CHUNK), jnp.int32),
            pltpu.VMEM((N_CHUNKS, CHUNK), jnp.int32),
            pltpu.VMEM((RING, CHUNK, width), jnp.float32),
            pltpu.VMEM((ROWS_T, width), jnp.float32),
            pltpu.VMEM_SHARED((N_PAD, width), jnp.float32),
            pltpu.SemaphoreType.DMA((RING,)),
            pltpu.SemaphoreType.DMA((RING,)),
        ],
        compiler_params=_SC_PARAMS,
    )


_agg20 = _make_agg(20)
_agg10 = _make_agg(10)


def _tc1_body(deg2_ref, x_ref, w1_ref, p1_ref, dinv_ref):
    deg = deg2_ref[0] + deg2_ref[1] + 1.0
    dinv = lax.rsqrt(deg)
    h = jnp.dot(x_ref[...], w1_ref[...], preferred_element_type=jnp.float32)
    p1_ref[...] = h * dinv
    dinv_ref[...] = dinv


def _tc2_body(acc_ref, p1_ref, dinv_ref, b1_ref, wmu_ref, p2_ref):
    q1 = acc_ref[0] + acc_ref[1] - p1_ref[...]
    h1 = jnp.maximum(q1 * dinv_ref[...] + b1_ref[...], 0.0)
    h2 = jnp.dot(h1, wmu_ref[...], preferred_element_type=jnp.float32)
    p2_ref[...] = h2 * dinv_ref[...]


def _tc3_body(acc_ref, p2_ref, dinv_ref, bmu_ref, z_ref):
    z_ref[...] = (acc_ref[0] + acc_ref[1] - p2_ref[...]) * dinv_ref[...] \
        + bmu_ref[...]


def _shard_edges(e):
    real = e.reshape(N_WORKERS, N_EDGES // N_WORKERS)
    pad = jnp.full((N_WORKERS, E_PER_W - N_EDGES // N_WORKERS), PAD_NODE,
                   jnp.int32)
    return jnp.concatenate([real, pad], axis=1).reshape(
        N_WORKERS, N_CHUNKS, CHUNK)


@jax.jit
def kernel(x, edge_index, W1, b1, W_mu, b_mu, W_ls, b_ls):
    del W_ls, b_ls
    ei = edge_index.astype(jnp.int32)
    src_r = _shard_edges(ei[0])
    dst_r = _shard_edges(ei[1])
    x_pad = jnp.zeros((N_PAD, x.shape[1]), x.dtype).at[:N_NODES].set(x)
    ones_c = jnp.ones((CHUNK, 1), jnp.float32)
    zero_t = jnp.zeros((ROWS_T, 1), jnp.float32)

    deg2 = _deg_kernel(dst_r, ones_c, zero_t)

    p1, dinv = pl.pallas_call(
        _tc1_body,
        out_shape=(jax.ShapeDtypeStruct((N_PAD, 20), jnp.float32),
                   jax.ShapeDtypeStruct((N_PAD, 1), jnp.float32)),
    )(deg2, x_pad, W1)

    acc1 = _agg20(src_r, dst_r, p1)

    p2 = pl.pallas_call(
        _tc2_body,
        out_shape=jax.ShapeDtypeStruct((N_PAD, 10), jnp.float32),
    )(acc1, p1, dinv, b1.reshape(1, 20), W_mu)

    acc2 = _agg10(src_r, dst_r, p2)

    z = pl.pallas_call(
        _tc3_body,
        out_shape=jax.ShapeDtypeStruct((N_PAD, 10), jnp.float32),
    )(acc2, p2, dinv, b_mu.reshape(1, 10))

    return z[:N_NODES]
